# Initial kernel scaffold; baseline (speedup 1.0000x reference)
import jax
import jax.numpy as jnp
from jax.experimental import pallas as pl


def kernel(x_openie, x_entity, ei_oe, ei_eo, ei_oo, Wp_o, bp_o, Wp_e, bp_e, att_src_oe, att_dst_oe, att_src_eo, att_dst_eo, att_src_oo, att_dst_oo, k_lin_W, k_lin_b, q, lin_W, lin_b):
    raise NotImplementedError("write your pallas kernel here")


# speedup vs baseline: 80.0313x; 80.0313x over previous
import functools

import jax
import jax.numpy as jnp
from jax import lax
from jax.experimental import pallas as pl
from jax.experimental.pallas import tpu as pltpu, tpu_sc as plsc

N = 10000
NP = 10240
C = 128
H = 8
DH = 16
E = 200000
B = 128
CHUNKS = 98
TILE_E = B * CHUNKS
EP = TILE_E * 16
STRIPE = NP // 16


def _zero16():
    return jnp.zeros((16,), jnp.float32)


def _edge_pass(sid, A, Bd, X, src, dst, cch, acc_out, ssum_out,
               src_v, dst_v, asrc_v, adst_v, ex_v, xrows_v, cc_v,
               acc_sh, ssum_sh, sem_a, sem_b, sem_x):
    pltpu.sync_copy(cch, cc_v)
    cc = cc_v[...]

    def chunk_body(j, carry):
        off = sid * TILE_E + j * B
        pltpu.sync_copy(src.at[pl.ds(off, B)], src_v)
        pltpu.sync_copy(dst.at[pl.ds(off, B)], dst_v)
        ca = pltpu.async_copy(A.at[src_v], asrc_v, sem_a)
        cb = pltpu.async_copy(Bd.at[dst_v], adst_v, sem_b)
        cx = pltpu.async_copy(X.at[src_v], xrows_v, sem_x)
        ca.wait()
        cb.wait()

        def ex_body(b, c2):
            v = asrc_v[b, :] + adst_v[b, :]
            v = jnp.where(v > 0.0, v, v * 0.2)
            ex_v[b, :] = jnp.exp(v - cc)
            return c2

        lax.fori_loop(0, B, ex_body, 0)
        cx.wait()

        def mul_body(b, c2):
            e_row = ex_v[b, :]
            for h in range(H):
                s = e_row[h]
                xrows_v[b, pl.ds(h * DH, DH)] = xrows_v[b, pl.ds(h * DH, DH)] * s
            return c2

        lax.fori_loop(0, B, mul_body, 0)
        pltpu.sync_copy(xrows_v, acc_sh.at[dst_v], add=True)
        pltpu.sync_copy(ex_v, ssum_sh.at[dst_v], add=True)
        return carry

    lax.fori_loop(0, CHUNKS, chunk_body, 0)
    plsc.subcore_barrier()
    for k in range(STRIPE // B):
        rows = pl.ds(sid * STRIPE + k * B, B)
        pltpu.sync_copy(acc_sh.at[rows], acc_out.at[rows])
        pltpu.sync_copy(ssum_sh.at[rows], ssum_out.at[rows])


def _sc_body(A_eo, Bd_eo, X_eo, src_eo, dst_eo, cc_eo,
             A_oo, Bd_oo, X_oo, src_oo, dst_oo, cc_oo,
             acc_eo, ssum_eo, acc_oo, ssum_oo,
             src_v, dst_v, asrc_v, adst_v, ex_v, xrows_v, cc_v,
             acc_sh, ssum_sh, sem_a, sem_b, sem_x):
    cid = lax.axis_index("c")
    sid = lax.axis_index("s")

    z = _zero16()

    def zero_body(b, carry):
        for h in range(H):
            xrows_v[b, pl.ds(h * DH, DH)] = z
        ex_v[b, :] = z
        return carry

    lax.fori_loop(0, B, zero_body, 0)
    for k in range(STRIPE // B):
        rows = pl.ds(sid * STRIPE + k * B, B)
        pltpu.sync_copy(xrows_v, acc_sh.at[rows])
        pltpu.sync_copy(ex_v, ssum_sh.at[rows])
    plsc.subcore_barrier()

    @pl.when(cid == 0)
    def _():
        _edge_pass(sid, A_eo, Bd_eo, X_eo, src_eo, dst_eo, cc_eo,
                   acc_eo, ssum_eo,
                   src_v, dst_v, asrc_v, adst_v, ex_v, xrows_v, cc_v,
                   acc_sh, ssum_sh, sem_a, sem_b, sem_x)

    @pl.when(cid == 1)
    def _():
        _edge_pass(sid, A_oo, Bd_oo, X_oo, src_oo, dst_oo, cc_oo,
                   acc_oo, ssum_oo,
                   src_v, dst_v, asrc_v, adst_v, ex_v, xrows_v, cc_v,
                   acc_sh, ssum_sh, sem_a, sem_b, sem_x)


@jax.jit
def _sc_call(A_eo, Bd_eo, X_eo, src_eo, dst_eo, cc_eo,
             A_oo, Bd_oo, X_oo, src_oo, dst_oo, cc_oo):
    mesh = plsc.VectorSubcoreMesh(core_axis_name="c", subcore_axis_name="s")
    f = pl.kernel(
        _sc_body,
        mesh=mesh,
        compiler_params=pltpu.CompilerParams(use_tc_tiling_on_sc=False),
        out_type=(
            jax.ShapeDtypeStruct((NP, C), jnp.float32),
            jax.ShapeDtypeStruct((NP, 16), jnp.float32),
            jax.ShapeDtypeStruct((NP, C), jnp.float32),
            jax.ShapeDtypeStruct((NP, 16), jnp.float32),
        ),
        scratch_types=[
            pltpu.VMEM((B,), jnp.int32),
            pltpu.VMEM((B,), jnp.int32),
            pltpu.VMEM((B, 16), jnp.float32),
            pltpu.VMEM((B, 16), jnp.float32),
            pltpu.VMEM((B, 16), jnp.float32),
            pltpu.VMEM((B, C), jnp.float32),
            pltpu.VMEM((16,), jnp.float32),
            pltpu.VMEM_SHARED((NP, C), jnp.float32),
            pltpu.VMEM_SHARED((NP, 16), jnp.float32),
            pltpu.SemaphoreType.DMA,
            pltpu.SemaphoreType.DMA,
            pltpu.SemaphoreType.DMA,
        ],
    )
    return f(A_eo, Bd_eo, X_eo, src_eo, dst_eo, cc_eo,
             A_oo, Bd_oo, X_oo, src_oo, dst_oo, cc_oo)


def _prep_type(x_src, asrc, adst, ei):
    c = asrc.max(0) + adst.max(0)
    cc = jnp.concatenate([c, jnp.zeros((8,), jnp.float32)])
    A = jnp.zeros((N + 1, 16), jnp.float32)
    A = A.at[:N, :H].set(asrc - 0.0)
    A = A.at[N, :].set(-1e30)
    Bd = jnp.zeros((N, 16), jnp.float32).at[:, :H].set(adst)
    X = jnp.concatenate([x_src, jnp.zeros((1, C), jnp.float32)], axis=0)
    src = jnp.full((EP,), N, jnp.int32).at[:E].set(ei[0].astype(jnp.int32))
    dst = jnp.zeros((EP,), jnp.int32).at[:E].set(ei[1].astype(jnp.int32))
    return A, Bd, X, src, dst, cc


def kernel(x_openie, x_entity, ei_oe, ei_eo, ei_oo, Wp_o, bp_o, Wp_e, bp_e,
           att_src_oe, att_dst_oe, att_src_eo, att_dst_eo, att_src_oo, att_dst_oo,
           k_lin_W, k_lin_b, q, lin_W, lin_b):
    x_o = x_openie @ Wp_o + bp_o
    x_e = x_entity @ Wp_e + bp_e

    def head_dot(x, att):
        return (x.reshape(N, H, DH) * att).sum(-1)

    asrc_eo = head_dot(x_e, att_src_eo)
    adst_eo = head_dot(x_o, att_dst_eo)
    asrc_oo = head_dot(x_o, att_src_oo)
    adst_oo = head_dot(x_o, att_dst_oo)

    args_eo = _prep_type(x_e, asrc_eo, adst_eo, ei_eo)
    args_oo = _prep_type(x_o, asrc_oo, adst_oo, ei_oo)

    acc_eo, ssum_eo, acc_oo, ssum_oo = _sc_call(*args_eo, *args_oo)

    def finish(acc, ssum):
        o = acc[:N].reshape(N, H, DH) / (ssum[:N, :H, None] + 1e-16)
        return jax.nn.relu(o).reshape(N, H * DH)

    out_eo = finish(acc_eo, ssum_eo)
    out_oo = finish(acc_oo, ssum_oo)

    out = jnp.stack([out_eo, out_oo])
    kk = jnp.tanh(out @ k_lin_W + k_lin_b).mean(axis=1)
    score = (q * kk).sum(-1)
    attn = jax.nn.softmax(score, axis=0)
    h_openie = (attn[:, None, None] * out).sum(0)
    pooled = h_openie.mean(axis=0, keepdims=True)
    return pooled @ lin_W + lin_b
